# baseline (speedup 1.0000x reference)
import jax
import jax.numpy as jnp
from jax import lax
from jax.experimental import pallas as pl
from jax.experimental.pallas import tpu as pltpu
from jax.experimental.pallas import tpu_sc as plsc

N_NODES = 10000
N_EDGES = 320000
D = 128

NC = 2
NS = 16
L = 16
NW = NC * NS
CHUNK = 64
NCH = 160
E_PAD = NW * NCH * CHUNK
QTR = 16
NBUF = 2
RPT = 624
TAIL = N_NODES - NS * RPT

_SELU_SCALE = 1.0507009873554805
_SELU_ALPHA = 1.6732632423543772


def _sc_agg_body(feat_hbm, rows_hbm, cols_hbm, vals_hbm, zeros_hbm, out_hbm,
                 cols_v, rows_v, vals_v, g0, g1, s0, s1, spmem_agg,
                 gs0, gs1, ss0, ss1):
    cid = lax.axis_index("c")
    sid = lax.axis_index("s")
    wid = cid * NS + sid

    zoff = pl.multiple_of(sid * RPT, 8)
    pltpu.sync_copy(zeros_hbm.at[pl.ds(zoff, RPT)],
                    spmem_agg.at[pl.ds(zoff, RPT)])
    @pl.when(sid == NS - 1)
    def _():
        pltpu.sync_copy(zeros_hbm.at[pl.ds(NS * RPT, TAIL)],
                        spmem_agg.at[pl.ds(NS * RPT, TAIL)])
    plsc.subcore_barrier()

    gbufs = (g0, g1)
    sbufs = (s0, s1)
    gsems = (gs0, gs1)
    ssems = (ss0, ss1)

    def start_gather(g, b):
        pltpu.async_copy(feat_hbm.at[cols_v.at[g]], gbufs[b], gsems[b])

    def wait_gather(b):
        pltpu.make_async_copy(feat_hbm.at[pl.ds(0, CHUNK)], gbufs[b],
                              gsems[b]).wait()

    def start_scatter(g, b):
        pltpu.async_copy(sbufs[b], spmem_agg.at[rows_v.at[g]], ssems[b],
                         add=True)

    def wait_scatter(b):
        pltpu.make_async_copy(feat_hbm.at[pl.ds(0, CHUNK)], sbufs[b],
                              ssems[b]).wait()

    def scale(g, b):
        gbuf = gbufs[b]
        sbuf = sbufs[b]

        def grp_body(k, c2):
            vgrp = vals_v[g, pl.ds(k * L, L)]
            for t in range(L):
                v = vgrp[t]
                e = k * L + t
                for j in range(D // L):
                    sl = pl.ds(j * L, L)
                    sbuf[e, sl] = gbuf[e, sl] * v
            return c2
        lax.fori_loop(0, CHUNK // L, grp_body, 0, unroll=False)

    def stage_body(h, c0):
        cbase = pl.multiple_of(wid * NCH + h * QTR, 8)
        pltpu.sync_copy(cols_hbm.at[pl.ds(cbase, QTR)], cols_v)
        pltpu.sync_copy(rows_hbm.at[pl.ds(cbase, QTR)], rows_v)
        pltpu.sync_copy(vals_hbm.at[pl.ds(cbase, QTR)], vals_v)

        start_gather(0, 0)
        start_gather(1, 1)

        for b in range(2):
            wait_gather(b)
            scale(b, b)
            start_gather(b + 2, b)
            start_scatter(b, b)

        def ring_body(i, c):
            ga = i * 2
            for b in range(2):
                g = ga + b
                wait_gather(b)
                wait_scatter(b)
                scale(g, b)
                start_gather(g + 2, b)
                start_scatter(g, b)
            return c
        lax.fori_loop(1, QTR // 2 - 1, ring_body, 0, unroll=False)

        for b in range(2):
            g = QTR - 2 + b
            wait_gather(b)
            wait_scatter(b)
            scale(g, b)
            start_scatter(g, b)
        for b in range(2):
            wait_scatter(b)
        return c0

    lax.fori_loop(0, NCH // QTR, stage_body, 0, unroll=False)

    plsc.subcore_barrier()

    woff = pl.multiple_of(sid * RPT, 8)
    pltpu.sync_copy(spmem_agg.at[pl.ds(woff, RPT)],
                    out_hbm.at[cid, pl.ds(woff, RPT)])
    @pl.when(sid == NS - 1)
    def _():
        pltpu.sync_copy(spmem_agg.at[pl.ds(NS * RPT, TAIL)],
                        out_hbm.at[cid, pl.ds(NS * RPT, TAIL)])


def _sc_aggregate(features, rows2, cols2, vals2, zeros):
    mesh = plsc.VectorSubcoreMesh(core_axis_name="c", subcore_axis_name="s")
    f = pl.kernel(
        _sc_agg_body,
        out_type=jax.ShapeDtypeStruct((NC, N_NODES, D), jnp.float32),
        mesh=mesh,
        scratch_types=[
            pltpu.VMEM((QTR, CHUNK), jnp.int32),
            pltpu.VMEM((QTR, CHUNK), jnp.int32),
            pltpu.VMEM((QTR, CHUNK), jnp.float32),
            pltpu.VMEM((CHUNK, D), jnp.float32),
            pltpu.VMEM((CHUNK, D), jnp.float32),
            pltpu.VMEM((CHUNK, D), jnp.float32),
            pltpu.VMEM((CHUNK, D), jnp.float32),
            pltpu.VMEM_SHARED((N_NODES, D), jnp.float32),
            pltpu.SemaphoreType.DMA,
            pltpu.SemaphoreType.DMA,
            pltpu.SemaphoreType.DMA,
            pltpu.SemaphoreType.DMA,
        ],
    )
    return f(features, rows2, cols2, vals2, zeros)


def _tc_body(f_ref, p_ref, k_ref, b_ref, sw_ref, o_ref):
    h = jnp.dot(f_ref[...], k_ref[...], preferred_element_type=jnp.float32,
                precision=lax.Precision.HIGHEST)
    agg = jnp.dot(p_ref[0] + p_ref[1], k_ref[...],
                  preferred_element_type=jnp.float32,
                  precision=lax.Precision.HIGHEST)
    y = h * sw_ref[...] + agg + b_ref[...]
    o_ref[...] = jnp.where(
        y > 0.0,
        _SELU_SCALE * y,
        (_SELU_SCALE * _SELU_ALPHA) * (jnp.exp(jnp.minimum(y, 0.0)) - 1.0),
    )


def _tc_finish(features, partials, k, bias2, sw2):
    BM = 2000
    return pl.pallas_call(
        _tc_body,
        grid=(N_NODES // BM,),
        in_specs=[
            pl.BlockSpec((BM, D), lambda i: (i, 0)),
            pl.BlockSpec((NC, BM, D), lambda i: (0, i, 0)),
            pl.BlockSpec((D, D), lambda i: (0, 0)),
            pl.BlockSpec((1, D), lambda i: (0, 0)),
            pl.BlockSpec((1, D), lambda i: (0, 0)),
        ],
        out_specs=pl.BlockSpec((BM, D), lambda i: (i, 0)),
        out_shape=jax.ShapeDtypeStruct((N_NODES, D), jnp.float32),
    )(features, partials, k, bias2, sw2)


def kernel(features, adj_indices, adj_values, kernel, bias, skip_weight):
    pad = E_PAD - N_EDGES
    idx2 = jnp.pad(adj_indices, ((0, 0), (0, pad)))
    rows2 = idx2[0].reshape(NW * NCH, CHUNK)
    cols2 = idx2[1].reshape(NW * NCH, CHUNK)
    vals2 = jnp.pad(adj_values, (0, pad)).reshape(NW * NCH, CHUNK)
    zeros = jnp.zeros((N_NODES, D), jnp.float32)
    partials = _sc_aggregate(features, rows2, cols2, vals2, zeros)
    return _tc_finish(features, partials, kernel,
                      bias.reshape(1, D), skip_weight.reshape(1, D))


# speedup vs baseline: 1.0155x; 1.0155x over previous
import jax
import jax.numpy as jnp
from jax import lax
from jax.experimental import pallas as pl
from jax.experimental.pallas import tpu as pltpu
from jax.experimental.pallas import tpu_sc as plsc

N_NODES = 10000
N_EDGES = 320000
D = 128

NC = 2
NS = 16
L = 16
NW = NC * NS
CHUNK = 64
NCH = 160
E_PAD = NW * NCH * CHUNK
RPT = 624
TAIL = N_NODES - NS * RPT

_SELU_SCALE = 1.0507009873554805
_SELU_ALPHA = 1.6732632423543772


def _sc_agg_body(feat_hbm, rows_hbm, cols_hbm, vals_hbm, zeros_hbm, out_hbm,
                 cols_v, rows_v, vals_v, g0, g1, s0, s1, spmem_agg,
                 gs0, gs1, ss0, ss1):
    cid = lax.axis_index("c")
    sid = lax.axis_index("s")
    wid = cid * NS + sid

    zoff = pl.multiple_of(sid * RPT, 8)
    pltpu.sync_copy(zeros_hbm.at[pl.ds(zoff, RPT)],
                    spmem_agg.at[pl.ds(zoff, RPT)])
    @pl.when(sid == NS - 1)
    def _():
        pltpu.sync_copy(zeros_hbm.at[pl.ds(NS * RPT, TAIL)],
                        spmem_agg.at[pl.ds(NS * RPT, TAIL)])
    plsc.subcore_barrier()

    gbufs = (g0, g1)
    sbufs = (s0, s1)
    gsems = (gs0, gs1)
    ssems = (ss0, ss1)

    def sync_idx(g, i4):
        ebase = pl.multiple_of(wid * NCH * CHUNK + g * CHUNK, 8)
        pltpu.sync_copy(cols_hbm.at[pl.ds(ebase, CHUNK)], cols_v.at[i4])
        pltpu.sync_copy(rows_hbm.at[pl.ds(ebase, CHUNK)], rows_v.at[i4])
        pltpu.sync_copy(vals_hbm.at[pl.ds(ebase, CHUNK)], vals_v.at[i4])

    def start_gather(i4, b):
        pltpu.async_copy(feat_hbm.at[cols_v.at[i4]], gbufs[b], gsems[b])

    def wait_gather(b):
        pltpu.make_async_copy(feat_hbm.at[pl.ds(0, CHUNK)], gbufs[b],
                              gsems[b]).wait()

    def start_scatter(i4, b):
        pltpu.async_copy(sbufs[b], spmem_agg.at[rows_v.at[i4]], ssems[b],
                         add=True)

    def wait_scatter(b):
        pltpu.make_async_copy(feat_hbm.at[pl.ds(0, CHUNK)], sbufs[b],
                              ssems[b]).wait()

    def scale(i4, b):
        gbuf = gbufs[b]
        sbuf = sbufs[b]

        def grp_body(k, c2):
            vgrp = vals_v[i4, pl.ds(k * L, L)]
            for t in range(L):
                v = vgrp[t]
                e = k * L + t
                for j in range(D // L):
                    sl = pl.ds(j * L, L)
                    sbuf[e, sl] = gbuf[e, sl] * v
            return c2
        lax.fori_loop(0, CHUNK // L, grp_body, 0, unroll=False)

    def body(g, i4, n4, b, first=False, last=False):
        wait_gather(b)
        if not first:
            wait_scatter(b)
        scale(i4, b)
        start_scatter(i4, b)
        if not last:
            if not first:
                sync_idx(g + 2, n4)
            start_gather(n4, b)

    for g in range(4):
        sync_idx(g, g)
    start_gather(0, 0)
    start_gather(1, 1)

    body(0, 0, 2, 0, first=True)
    body(1, 1, 3, 1, first=True)
    body(2, 2, 0, 0)
    body(3, 3, 1, 1)

    def ring_body(i, c):
        ga = i * 4
        body(ga + 0, 0, 2, 0)
        body(ga + 1, 1, 3, 1)
        body(ga + 2, 2, 0, 0)
        body(ga + 3, 3, 1, 1)
        return c
    lax.fori_loop(1, NCH // 4 - 1, ring_body, 0, unroll=False)

    body(NCH - 4, 0, 2, 0)
    body(NCH - 3, 1, 3, 1)
    body(NCH - 2, 2, 0, 0, last=True)
    body(NCH - 1, 3, 1, 1, last=True)
    wait_scatter(0)
    wait_scatter(1)

    plsc.subcore_barrier()

    woff = pl.multiple_of(sid * RPT, 8)
    pltpu.sync_copy(spmem_agg.at[pl.ds(woff, RPT)],
                    out_hbm.at[cid, pl.ds(woff, RPT)])
    @pl.when(sid == NS - 1)
    def _():
        pltpu.sync_copy(spmem_agg.at[pl.ds(NS * RPT, TAIL)],
                        out_hbm.at[cid, pl.ds(NS * RPT, TAIL)])


def _sc_aggregate(features, rows1, cols1, vals1, zeros):
    mesh = plsc.VectorSubcoreMesh(core_axis_name="c", subcore_axis_name="s")
    f = pl.kernel(
        _sc_agg_body,
        out_type=jax.ShapeDtypeStruct((NC, N_NODES, D), jnp.float32),
        mesh=mesh,
        scratch_types=[
            pltpu.VMEM((4, CHUNK), jnp.int32),
            pltpu.VMEM((4, CHUNK), jnp.int32),
            pltpu.VMEM((4, CHUNK), jnp.float32),
            pltpu.VMEM((CHUNK, D), jnp.float32),
            pltpu.VMEM((CHUNK, D), jnp.float32),
            pltpu.VMEM((CHUNK, D), jnp.float32),
            pltpu.VMEM((CHUNK, D), jnp.float32),
            pltpu.VMEM_SHARED((N_NODES, D), jnp.float32),
            pltpu.SemaphoreType.DMA,
            pltpu.SemaphoreType.DMA,
            pltpu.SemaphoreType.DMA,
            pltpu.SemaphoreType.DMA,
        ],
    )
    return f(features, rows1, cols1, vals1, zeros)


def _tc_body(f_ref, p_ref, k_ref, b_ref, sw_ref, o_ref):
    h = jnp.dot(f_ref[...], k_ref[...], preferred_element_type=jnp.float32,
                precision=lax.Precision.HIGHEST)
    agg = jnp.dot(p_ref[0] + p_ref[1], k_ref[...],
                  preferred_element_type=jnp.float32,
                  precision=lax.Precision.HIGHEST)
    y = h * sw_ref[...] + agg + b_ref[...]
    o_ref[...] = jnp.where(
        y > 0.0,
        _SELU_SCALE * y,
        (_SELU_SCALE * _SELU_ALPHA) * (jnp.exp(jnp.minimum(y, 0.0)) - 1.0),
    )


def _tc_finish(features, partials, k, bias2, sw2):
    BM = 2000
    return pl.pallas_call(
        _tc_body,
        grid=(N_NODES // BM,),
        in_specs=[
            pl.BlockSpec((BM, D), lambda i: (i, 0)),
            pl.BlockSpec((NC, BM, D), lambda i: (0, i, 0)),
            pl.BlockSpec((D, D), lambda i: (0, 0)),
            pl.BlockSpec((1, D), lambda i: (0, 0)),
            pl.BlockSpec((1, D), lambda i: (0, 0)),
        ],
        out_specs=pl.BlockSpec((BM, D), lambda i: (i, 0)),
        out_shape=jax.ShapeDtypeStruct((N_NODES, D), jnp.float32),
    )(features, partials, k, bias2, sw2)


def kernel(features, adj_indices, adj_values, kernel, bias, skip_weight):
    pad = E_PAD - N_EDGES
    idx2 = jnp.pad(adj_indices, ((0, 0), (0, pad)))
    rows1 = idx2[0]
    cols1 = idx2[1]
    vals1 = jnp.pad(adj_values, (0, pad))
    zeros = jnp.zeros((N_NODES, D), jnp.float32)
    partials = _sc_aggregate(features, rows1, cols1, vals1, zeros)
    return _tc_finish(features, partials, kernel,
                      bias.reshape(1, D), skip_weight.reshape(1, D))


# speedup vs baseline: 1.2348x; 1.2159x over previous
import functools

import jax
import jax.numpy as jnp
from jax import lax
from jax.experimental import pallas as pl
from jax.experimental.pallas import tpu as pltpu
from jax.experimental.pallas import tpu_sc as plsc

N_NODES = 10000
N_EDGES = 320000
D = 128

NC = 2
NS = 16
L = 16
NW = NC * NS
EPW = N_EDGES // NW
CHUNK = 80
NCHUNK = EPW // CHUNK
RPT = 624
TAIL = N_NODES - NS * RPT

_SELU_SCALE = 1.0507009873554805
_SELU_ALPHA = 1.6732632423543772


def _sc_agg_body(feat_hbm, rows_hbm, cv_hbm, zeros_hbm, out_hbm,
                 cv_v, rows_v, gath_v, spmem_agg, gsem):
    cid = lax.axis_index("c")
    sid = lax.axis_index("s")
    wid = cid * NS + sid

    zoff = pl.multiple_of(sid * RPT, 8)
    pltpu.sync_copy(zeros_hbm.at[pl.ds(zoff, RPT)],
                    spmem_agg.at[pl.ds(zoff, RPT)])
    @pl.when(sid == NS - 1)
    def _():
        pltpu.sync_copy(zeros_hbm.at[pl.ds(NS * RPT, TAIL)],
                        spmem_agg.at[pl.ds(NS * RPT, TAIL)])
    plsc.subcore_barrier()

    def chunk_body(g, carry):
        ebase = pl.multiple_of(wid * EPW + g * CHUNK, 8)
        cvbase = pl.multiple_of(2 * (wid * EPW + g * CHUNK), 8)
        pltpu.sync_copy(cv_hbm.at[pl.ds(cvbase, 2 * CHUNK)], cv_v)
        pltpu.sync_copy(rows_hbm.at[pl.ds(ebase, CHUNK)], rows_v)
        pltpu.async_copy(feat_hbm.at[cv_v.at[pl.ds(0, CHUNK)]], gath_v,
                         gsem).wait()

        def grp_body(k, c2):
            vals_grp = lax.bitcast_convert_type(
                cv_v[pl.ds(CHUNK + k * L, L)], jnp.float32)
            for t in range(L):
                v = vals_grp[t]
                e = k * L + t
                for j in range(D // L):
                    sl = pl.ds(j * L, L)
                    gath_v[e, sl] = gath_v[e, sl] * v
            return c2
        lax.fori_loop(0, CHUNK // L, grp_body, 0, unroll=False)

        pltpu.sync_copy(gath_v, spmem_agg.at[rows_v], add=True)
        return carry

    lax.fori_loop(0, NCHUNK, chunk_body, 0, unroll=False)
    plsc.subcore_barrier()

    woff = pl.multiple_of(sid * RPT, 8)
    pltpu.sync_copy(spmem_agg.at[pl.ds(woff, RPT)],
                    out_hbm.at[cid, pl.ds(woff, RPT)])
    @pl.when(sid == NS - 1)
    def _():
        pltpu.sync_copy(spmem_agg.at[pl.ds(NS * RPT, TAIL)],
                        out_hbm.at[cid, pl.ds(NS * RPT, TAIL)])


def _sc_aggregate(features, rows, colvals, zeros):
    mesh = plsc.VectorSubcoreMesh(core_axis_name="c", subcore_axis_name="s")
    f = pl.kernel(
        _sc_agg_body,
        out_type=jax.ShapeDtypeStruct((NC, N_NODES, D), jnp.float32),
        mesh=mesh,
        scratch_types=[
            pltpu.VMEM((2 * CHUNK,), jnp.int32),
            pltpu.VMEM((CHUNK,), jnp.int32),
            pltpu.VMEM((CHUNK, D), jnp.float32),
            pltpu.VMEM_SHARED((N_NODES, D), jnp.float32),
            pltpu.SemaphoreType.DMA,
        ],
    )
    return f(features, rows, colvals, zeros)


def _tc_body(f_ref, p_ref, k_ref, b_ref, sw_ref, o_ref):
    h = jnp.dot(f_ref[...], k_ref[...], preferred_element_type=jnp.float32,
                precision=lax.Precision.HIGHEST)
    agg = jnp.dot(p_ref[0] + p_ref[1], k_ref[...],
                  preferred_element_type=jnp.float32,
                  precision=lax.Precision.HIGHEST)
    y = h * sw_ref[...] + agg + b_ref[...]
    o_ref[...] = jnp.where(
        y > 0.0,
        _SELU_SCALE * y,
        (_SELU_SCALE * _SELU_ALPHA) * (jnp.exp(jnp.minimum(y, 0.0)) - 1.0),
    )


def _tc_finish(features, partials, k, bias2, sw2):
    BM = 2000
    return pl.pallas_call(
        _tc_body,
        grid=(N_NODES // BM,),
        in_specs=[
            pl.BlockSpec((BM, D), lambda i: (i, 0)),
            pl.BlockSpec((NC, BM, D), lambda i: (0, i, 0)),
            pl.BlockSpec((D, D), lambda i: (0, 0)),
            pl.BlockSpec((1, D), lambda i: (0, 0)),
            pl.BlockSpec((1, D), lambda i: (0, 0)),
        ],
        out_specs=pl.BlockSpec((BM, D), lambda i: (i, 0)),
        out_shape=jax.ShapeDtypeStruct((N_NODES, D), jnp.float32),
    )(features, partials, k, bias2, sw2)


def kernel(features, adj_indices, adj_values, kernel, bias, skip_weight):
    rows = adj_indices[0]
    cols2 = adj_indices[1].reshape(-1, CHUNK)
    vals2 = lax.bitcast_convert_type(adj_values, jnp.int32).reshape(-1, CHUNK)
    colvals = jnp.stack([cols2, vals2], axis=1).reshape(-1)
    zeros = jnp.zeros((N_NODES, D), jnp.float32)
    partials = _sc_aggregate(features, rows, colvals, zeros)
    return _tc_finish(features, partials, kernel,
                      bias.reshape(1, D), skip_weight.reshape(1, D))


# speedup vs baseline: 1.4065x; 1.1390x over previous
import functools

import jax
import jax.numpy as jnp
from jax import lax
from jax.experimental import pallas as pl
from jax.experimental.pallas import tpu as pltpu
from jax.experimental.pallas import tpu_sc as plsc

N_NODES = 10000
N_EDGES = 320000
D = 128

NC = 2
NS = 16
L = 16
NW = NC * NS
EPW = N_EDGES // NW
CHUNK = 80
NCHUNK = EPW // CHUNK
RPT = 624
TAIL = N_NODES - NS * RPT

_SELU_SCALE = 1.0507009873554805
_SELU_ALPHA = 1.6732632423543772


def _sc_agg_body(feat_hbm, rows_hbm, cv_hbm, zeros_hbm, out_hbm,
                 cv_v, rows_v, gath_v, spmem_agg, gsem):
    cid = lax.axis_index("c")
    sid = lax.axis_index("s")
    wid = cid * NS + sid

    zoff = pl.multiple_of(sid * RPT, 8)
    pltpu.sync_copy(zeros_hbm.at[pl.ds(zoff, RPT)],
                    spmem_agg.at[pl.ds(zoff, RPT)])
    @pl.when(sid == NS - 1)
    def _():
        pltpu.sync_copy(zeros_hbm.at[pl.ds(NS * RPT, TAIL)],
                        spmem_agg.at[pl.ds(NS * RPT, TAIL)])
    plsc.subcore_barrier()

    def chunk_body(g, carry):
        ebase = pl.multiple_of(wid * EPW + g * CHUNK, 8)
        cvbase = pl.multiple_of(2 * (wid * EPW + g * CHUNK), 8)
        pltpu.sync_copy(cv_hbm.at[pl.ds(cvbase, 2 * CHUNK)], cv_v)
        gather = pltpu.async_copy(feat_hbm.at[cv_v.at[pl.ds(0, CHUNK)]],
                                  gath_v, gsem)
        pltpu.sync_copy(rows_hbm.at[pl.ds(ebase, CHUNK)], rows_v)
        gather.wait()

        def grp_body(k, c2):
            vals_grp = lax.bitcast_convert_type(
                cv_v[pl.ds(CHUNK + k * L, L)], jnp.float32)
            for t in range(L):
                v = vals_grp[t]
                e = k * L + t
                for j in range(D // L):
                    sl = pl.ds(j * L, L)
                    gath_v[e, sl] = gath_v[e, sl] * v
            return c2
        lax.fori_loop(0, CHUNK // L, grp_body, 0, unroll=False)

        pltpu.sync_copy(gath_v, spmem_agg.at[rows_v], add=True)
        return carry

    lax.fori_loop(0, NCHUNK, chunk_body, 0, unroll=False)
    plsc.subcore_barrier()

    woff = pl.multiple_of(sid * RPT, 8)
    pltpu.sync_copy(spmem_agg.at[pl.ds(woff, RPT)],
                    out_hbm.at[cid, pl.ds(woff, RPT)])
    @pl.when(sid == NS - 1)
    def _():
        pltpu.sync_copy(spmem_agg.at[pl.ds(NS * RPT, TAIL)],
                        out_hbm.at[cid, pl.ds(NS * RPT, TAIL)])


def _sc_aggregate(features, rows, colvals, zeros):
    mesh = plsc.VectorSubcoreMesh(core_axis_name="c", subcore_axis_name="s")
    f = pl.kernel(
        _sc_agg_body,
        out_type=jax.ShapeDtypeStruct((NC, N_NODES, D), jnp.float32),
        mesh=mesh,
        scratch_types=[
            pltpu.VMEM((2 * CHUNK,), jnp.int32),
            pltpu.VMEM((CHUNK,), jnp.int32),
            pltpu.VMEM((CHUNK, D), jnp.float32),
            pltpu.VMEM_SHARED((N_NODES, D), jnp.float32),
            pltpu.SemaphoreType.DMA,
        ],
    )
    return f(features, rows, colvals, zeros)


def _tc_body(f_ref, p_ref, k_ref, b_ref, sw_ref, o_ref):
    h = jnp.dot(f_ref[...], k_ref[...], preferred_element_type=jnp.float32,
                precision=lax.Precision.HIGHEST)
    agg = jnp.dot(p_ref[0] + p_ref[1], k_ref[...],
                  preferred_element_type=jnp.float32,
                  precision=lax.Precision.HIGHEST)
    y = h * sw_ref[...] + agg + b_ref[...]
    o_ref[...] = jnp.where(
        y > 0.0,
        _SELU_SCALE * y,
        (_SELU_SCALE * _SELU_ALPHA) * (jnp.exp(jnp.minimum(y, 0.0)) - 1.0),
    )


def _tc_finish(features, partials, k, bias2, sw2):
    BM = 2000
    return pl.pallas_call(
        _tc_body,
        grid=(N_NODES // BM,),
        in_specs=[
            pl.BlockSpec((BM, D), lambda i: (i, 0)),
            pl.BlockSpec((NC, BM, D), lambda i: (0, i, 0)),
            pl.BlockSpec((D, D), lambda i: (0, 0)),
            pl.BlockSpec((1, D), lambda i: (0, 0)),
            pl.BlockSpec((1, D), lambda i: (0, 0)),
        ],
        out_specs=pl.BlockSpec((BM, D), lambda i: (i, 0)),
        out_shape=jax.ShapeDtypeStruct((N_NODES, D), jnp.float32),
    )(features, partials, k, bias2, sw2)


def kernel(features, adj_indices, adj_values, kernel, bias, skip_weight):
    rows = adj_indices[0]
    cols2 = adj_indices[1].reshape(-1, CHUNK)
    vals2 = lax.bitcast_convert_type(adj_values, jnp.int32).reshape(-1, CHUNK)
    colvals = jnp.stack([cols2, vals2], axis=1).reshape(-1)
    zeros = jnp.zeros((N_NODES, D), jnp.float32)
    partials = _sc_aggregate(features, rows, colvals, zeros)
    return _tc_finish(features, partials, kernel,
                      bias.reshape(1, D), skip_weight.reshape(1, D))


# speedup vs baseline: 1.6351x; 1.1626x over previous
import functools

import jax
import jax.numpy as jnp
from jax import lax
from jax.experimental import pallas as pl
from jax.experimental.pallas import tpu as pltpu
from jax.experimental.pallas import tpu_sc as plsc

N_NODES = 10000
N_EDGES = 320000
D = 128

NC = 2
NS = 16
L = 16
NW = NC * NS
EPW = N_EDGES // NW
CHUNK = 80
NCHUNK = EPW // CHUNK
RPT = 624
TAIL = N_NODES - NS * RPT

_SELU_SCALE = 1.0507009873554805
_SELU_ALPHA = 1.6732632423543772


def _sc_agg_body(feat_hbm, rows_hbm, cv_hbm, zeros_hbm, out_hbm,
                 cv0, cv1, rows0, rows1, gath_v, spmem_agg, gsem, is0, is1):
    cid = lax.axis_index("c")
    sid = lax.axis_index("s")
    wid = cid * NS + sid

    zoff = pl.multiple_of(sid * RPT, 8)
    pltpu.sync_copy(zeros_hbm.at[pl.ds(zoff, RPT)],
                    spmem_agg.at[pl.ds(zoff, RPT)])
    @pl.when(sid == NS - 1)
    def _():
        pltpu.sync_copy(zeros_hbm.at[pl.ds(NS * RPT, TAIL)],
                        spmem_agg.at[pl.ds(NS * RPT, TAIL)])
    plsc.subcore_barrier()

    cvs = (cv0, cv1)
    rows = (rows0, rows1)
    isems = (is0, is1)

    def start_idx(g, b):
        ebase = pl.multiple_of(wid * EPW + g * CHUNK, 8)
        cvbase = pl.multiple_of(2 * (wid * EPW + g * CHUNK), 8)
        pltpu.async_copy(cv_hbm.at[pl.ds(cvbase, 2 * CHUNK)], cvs[b],
                         isems[b])
        pltpu.async_copy(rows_hbm.at[pl.ds(ebase, CHUNK)], rows[b],
                         isems[b])

    def wait_idx(b):
        pltpu.make_async_copy(cv_hbm.at[pl.ds(0, 2 * CHUNK)], cvs[b],
                              isems[b]).wait()
        pltpu.make_async_copy(rows_hbm.at[pl.ds(0, CHUNK)], rows[b],
                              isems[b]).wait()

    def process(g, b, prefetch):
        gather = pltpu.async_copy(feat_hbm.at[cvs[b].at[pl.ds(0, CHUNK)]],
                                  gath_v, gsem)
        if prefetch:
            start_idx(g + 1, 1 - b)
        gather.wait()

        def grp_body(k, c2):
            vals_grp = lax.bitcast_convert_type(
                cvs[b][pl.ds(CHUNK + k * L, L)], jnp.float32)
            for t in range(L):
                v = vals_grp[t]
                e = k * L + t
                for j in range(D // L):
                    sl = pl.ds(j * L, L)
                    gath_v[e, sl] = gath_v[e, sl] * v
            return c2
        lax.fori_loop(0, CHUNK // L, grp_body, 0, unroll=False)

        pltpu.sync_copy(gath_v, spmem_agg.at[rows[b]], add=True)
        if prefetch:
            wait_idx(1 - b)

    start_idx(0, 0)
    wait_idx(0)

    def pair_body(i, carry):
        ga = i * 2
        process(ga, 0, True)
        process(ga + 1, 1, True)
        return carry
    lax.fori_loop(0, NCHUNK // 2, pair_body, 0, unroll=False)

    process(NCHUNK - 1, 0, False)
    plsc.subcore_barrier()

    woff = pl.multiple_of(sid * RPT, 8)
    pltpu.sync_copy(spmem_agg.at[pl.ds(woff, RPT)],
                    out_hbm.at[cid, pl.ds(woff, RPT)])
    @pl.when(sid == NS - 1)
    def _():
        pltpu.sync_copy(spmem_agg.at[pl.ds(NS * RPT, TAIL)],
                        out_hbm.at[cid, pl.ds(NS * RPT, TAIL)])


def _sc_aggregate(features, rows, colvals, zeros):
    mesh = plsc.VectorSubcoreMesh(core_axis_name="c", subcore_axis_name="s")
    f = pl.kernel(
        _sc_agg_body,
        out_type=jax.ShapeDtypeStruct((NC, N_NODES, D), jnp.float32),
        mesh=mesh,
        scratch_types=[
            pltpu.VMEM((2 * CHUNK,), jnp.int32),
            pltpu.VMEM((2 * CHUNK,), jnp.int32),
            pltpu.VMEM((CHUNK,), jnp.int32),
            pltpu.VMEM((CHUNK,), jnp.int32),
            pltpu.VMEM((CHUNK, D), jnp.float32),
            pltpu.VMEM_SHARED((N_NODES, D), jnp.float32),
            pltpu.SemaphoreType.DMA,
            pltpu.SemaphoreType.DMA,
            pltpu.SemaphoreType.DMA,
        ],
    )
    return f(features, rows, colvals, zeros)


def _tc_body(f_ref, p_ref, k_ref, b_ref, sw_ref, o_ref):
    h = jnp.dot(f_ref[...], k_ref[...], preferred_element_type=jnp.float32,
                precision=lax.Precision.HIGHEST)
    agg = jnp.dot(p_ref[0] + p_ref[1], k_ref[...],
                  preferred_element_type=jnp.float32,
                  precision=lax.Precision.HIGHEST)
    y = h * sw_ref[...] + agg + b_ref[...]
    o_ref[...] = jnp.where(
        y > 0.0,
        _SELU_SCALE * y,
        (_SELU_SCALE * _SELU_ALPHA) * (jnp.exp(jnp.minimum(y, 0.0)) - 1.0),
    )


def _tc_finish(features, partials, k, bias2, sw2):
    BM = 2000
    return pl.pallas_call(
        _tc_body,
        grid=(N_NODES // BM,),
        in_specs=[
            pl.BlockSpec((BM, D), lambda i: (i, 0)),
            pl.BlockSpec((NC, BM, D), lambda i: (0, i, 0)),
            pl.BlockSpec((D, D), lambda i: (0, 0)),
            pl.BlockSpec((1, D), lambda i: (0, 0)),
            pl.BlockSpec((1, D), lambda i: (0, 0)),
        ],
        out_specs=pl.BlockSpec((BM, D), lambda i: (i, 0)),
        out_shape=jax.ShapeDtypeStruct((N_NODES, D), jnp.float32),
    )(features, partials, k, bias2, sw2)


def kernel(features, adj_indices, adj_values, kernel, bias, skip_weight):
    rows = adj_indices[0]
    cols2 = adj_indices[1].reshape(-1, CHUNK)
    vals2 = lax.bitcast_convert_type(adj_values, jnp.int32).reshape(-1, CHUNK)
    colvals = jnp.stack([cols2, vals2], axis=1).reshape(-1)
    zeros = jnp.zeros((N_NODES, D), jnp.float32)
    partials = _sc_aggregate(features, rows, colvals, zeros)
    return _tc_finish(features, partials, kernel,
                      bias.reshape(1, D), skip_weight.reshape(1, D))


# speedup vs baseline: 1.9316x; 1.1813x over previous
import functools

import jax
import jax.numpy as jnp
from jax import lax
from jax.experimental import pallas as pl
from jax.experimental.pallas import tpu as pltpu
from jax.experimental.pallas import tpu_sc as plsc

N_NODES = 10000
N_EDGES = 320000
D = 128

NC = 2
NS = 16
L = 16
NW = NC * NS
EPW = N_EDGES // NW
CHUNK = 80
NCHUNK = EPW // CHUNK
RPT = 624
TAIL = N_NODES - NS * RPT

_SELU_SCALE = 1.0507009873554805
_SELU_ALPHA = 1.6732632423543772


def _sc_agg_body(feat_hbm, rows_hbm, cv_hbm, zeros_hbm, out_hbm,
                 cv0, cv1, rows0, rows1, ga0, ga1, spmem_agg,
                 gs0, gs1, is0, is1, ss0, ss1):
    cid = lax.axis_index("c")
    sid = lax.axis_index("s")
    wid = cid * NS + sid

    zoff = pl.multiple_of(sid * RPT, 8)
    pltpu.sync_copy(zeros_hbm.at[pl.ds(zoff, RPT)],
                    spmem_agg.at[pl.ds(zoff, RPT)])
    @pl.when(sid == NS - 1)
    def _():
        pltpu.sync_copy(zeros_hbm.at[pl.ds(NS * RPT, TAIL)],
                        spmem_agg.at[pl.ds(NS * RPT, TAIL)])
    plsc.subcore_barrier()

    cvs = (cv0, cv1)
    rows = (rows0, rows1)
    gaths = (ga0, ga1)
    gsems = (gs0, gs1)
    isems = (is0, is1)
    ssems = (ss0, ss1)

    def start_idx(g, b):
        ebase = pl.multiple_of(wid * EPW + g * CHUNK, 8)
        cvbase = pl.multiple_of(2 * (wid * EPW + g * CHUNK), 8)
        pltpu.async_copy(cv_hbm.at[pl.ds(cvbase, 2 * CHUNK)], cvs[b],
                         isems[b])
        pltpu.async_copy(rows_hbm.at[pl.ds(ebase, CHUNK)], rows[b],
                         isems[b])

    def wait_idx(b):
        pltpu.make_async_copy(cv_hbm.at[pl.ds(0, 2 * CHUNK)], cvs[b],
                              isems[b]).wait()
        pltpu.make_async_copy(rows_hbm.at[pl.ds(0, CHUNK)], rows[b],
                              isems[b]).wait()

    def wait_scatter(b):
        pltpu.make_async_copy(feat_hbm.at[pl.ds(0, CHUNK)], gaths[b],
                              ssems[b]).wait()

    def process(g, b, prefetch, first=False):
        if not first:
            wait_scatter(b)
        gather = pltpu.async_copy(feat_hbm.at[cvs[b].at[pl.ds(0, CHUNK)]],
                                  gaths[b], gsems[b])
        if prefetch:
            start_idx(g + 1, 1 - b)
        gather.wait()

        def grp_body(k, c2):
            vals_grp = lax.bitcast_convert_type(
                cvs[b][pl.ds(CHUNK + k * L, L)], jnp.float32)
            for t in range(L):
                v = vals_grp[t]
                e = k * L + t
                for j in range(D // L):
                    sl = pl.ds(j * L, L)
                    gaths[b][e, sl] = gaths[b][e, sl] * v
            return c2
        lax.fori_loop(0, CHUNK // L, grp_body, 0, unroll=False)

        pltpu.async_copy(gaths[b], spmem_agg.at[rows[b]], ssems[b],
                         add=True)
        if prefetch:
            wait_idx(1 - b)

    start_idx(0, 0)
    wait_idx(0)

    process(0, 0, True, first=True)
    process(1, 1, True, first=True)

    def pair_body(i, carry):
        ga = i * 2
        process(ga, 0, True)
        process(ga + 1, 1, True)
        return carry
    lax.fori_loop(1, NCHUNK // 2, pair_body, 0, unroll=False)

    process(NCHUNK - 1, 0, False)
    wait_scatter(0)
    wait_scatter(1)
    plsc.subcore_barrier()

    woff = pl.multiple_of(sid * RPT, 8)
    pltpu.sync_copy(spmem_agg.at[pl.ds(woff, RPT)],
                    out_hbm.at[cid, pl.ds(woff, RPT)])
    @pl.when(sid == NS - 1)
    def _():
        pltpu.sync_copy(spmem_agg.at[pl.ds(NS * RPT, TAIL)],
                        out_hbm.at[cid, pl.ds(NS * RPT, TAIL)])


def _sc_aggregate(features, rows, colvals, zeros):
    mesh = plsc.VectorSubcoreMesh(core_axis_name="c", subcore_axis_name="s")
    f = pl.kernel(
        _sc_agg_body,
        out_type=jax.ShapeDtypeStruct((NC, N_NODES, D), jnp.float32),
        mesh=mesh,
        scratch_types=[
            pltpu.VMEM((2 * CHUNK,), jnp.int32),
            pltpu.VMEM((2 * CHUNK,), jnp.int32),
            pltpu.VMEM((CHUNK,), jnp.int32),
            pltpu.VMEM((CHUNK,), jnp.int32),
            pltpu.VMEM((CHUNK, D), jnp.float32),
            pltpu.VMEM((CHUNK, D), jnp.float32),
            pltpu.VMEM_SHARED((N_NODES, D), jnp.float32),
            pltpu.SemaphoreType.DMA,
            pltpu.SemaphoreType.DMA,
            pltpu.SemaphoreType.DMA,
            pltpu.SemaphoreType.DMA,
            pltpu.SemaphoreType.DMA,
            pltpu.SemaphoreType.DMA,
        ],
    )
    return f(features, rows, colvals, zeros)


def _tc_body(f_ref, p_ref, k_ref, b_ref, sw_ref, o_ref):
    h = jnp.dot(f_ref[...], k_ref[...], preferred_element_type=jnp.float32,
                precision=lax.Precision.HIGHEST)
    agg = jnp.dot(p_ref[0] + p_ref[1], k_ref[...],
                  preferred_element_type=jnp.float32,
                  precision=lax.Precision.HIGHEST)
    y = h * sw_ref[...] + agg + b_ref[...]
    o_ref[...] = jnp.where(
        y > 0.0,
        _SELU_SCALE * y,
        (_SELU_SCALE * _SELU_ALPHA) * (jnp.exp(jnp.minimum(y, 0.0)) - 1.0),
    )


def _tc_finish(features, partials, k, bias2, sw2):
    BM = 2000
    return pl.pallas_call(
        _tc_body,
        grid=(N_NODES // BM,),
        in_specs=[
            pl.BlockSpec((BM, D), lambda i: (i, 0)),
            pl.BlockSpec((NC, BM, D), lambda i: (0, i, 0)),
            pl.BlockSpec((D, D), lambda i: (0, 0)),
            pl.BlockSpec((1, D), lambda i: (0, 0)),
            pl.BlockSpec((1, D), lambda i: (0, 0)),
        ],
        out_specs=pl.BlockSpec((BM, D), lambda i: (i, 0)),
        out_shape=jax.ShapeDtypeStruct((N_NODES, D), jnp.float32),
    )(features, partials, k, bias2, sw2)


def kernel(features, adj_indices, adj_values, kernel, bias, skip_weight):
    rows = adj_indices[0]
    cols2 = adj_indices[1].reshape(-1, CHUNK)
    vals2 = lax.bitcast_convert_type(adj_values, jnp.int32).reshape(-1, CHUNK)
    colvals = jnp.stack([cols2, vals2], axis=1).reshape(-1)
    zeros = jnp.zeros((N_NODES, D), jnp.float32)
    partials = _sc_aggregate(features, rows, colvals, zeros)
    return _tc_finish(features, partials, kernel,
                      bias.reshape(1, D), skip_weight.reshape(1, D))
